# baseline (speedup 1.0000x reference)
import jax
import jax.numpy as jnp
from jax.experimental import pallas as pl
from jax.experimental.pallas import tpu as pltpu

_N = 1_000_000
_D = 64
_NCHUNK = 8
_CHUNK = _N // _NCHUNK


def _dma_body(x_hbm, o_hbm, two_vmem, copy_sems, row_sem):
    two_vmem[...] = jnp.full((8, _D), 2.0, jnp.float32)
    copies = []
    cp = pltpu.make_async_copy(x_hbm, o_hbm, copy_sems.at[0])
    cp.start()
    copies.append(cp)
    copies[0].wait()
    r1 = pltpu.make_async_copy(
        two_vmem.at[pl.ds(0, 1), :], o_hbm.at[pl.ds(1, 1), :], row_sem)
    r1.start()
    r3 = pltpu.make_async_copy(
        two_vmem.at[pl.ds(0, 1), :], o_hbm.at[pl.ds(3, 1), :], row_sem)
    r1.wait()
    r3.start()
    r3.wait()
    for cp in copies[1:]:
        cp.wait()


def kernel(x):
    return pl.pallas_call(
        _dma_body,
        in_specs=[pl.BlockSpec(memory_space=pl.ANY)],
        out_specs=pl.BlockSpec(memory_space=pl.ANY),
        out_shape=jax.ShapeDtypeStruct((_N, _D), jnp.float32),
        scratch_shapes=[
            pltpu.VMEM((8, _D), jnp.float32),
            pltpu.SemaphoreType.DMA((_NCHUNK,)),
            pltpu.SemaphoreType.DMA,
        ],
    )(x)


# speedup vs baseline: 11.8265x; 11.8265x over previous
import jax
import jax.numpy as jnp
from jax.experimental import pallas as pl

_N = 1_000_000
_D = 64
_NV = _N // 2
_DV = 128
_BLOCK = 10_000


def _copy_body(x_ref, o_ref):
    o_ref[...] = x_ref[...]

    @pl.when(pl.program_id(0) == 0)
    def _():
        two = jnp.full((2, _DV // 2), 2.0, jnp.float32)
        o_ref[pl.ds(0, 2), pl.ds(_DV // 2, _DV // 2)] = two


def kernel(x):
    xv = x.reshape(_NV, _DV)
    out = pl.pallas_call(
        _copy_body,
        grid=(_NV // _BLOCK,),
        in_specs=[pl.BlockSpec((_BLOCK, _DV), lambda i: (i, 0))],
        out_specs=pl.BlockSpec((_BLOCK, _DV), lambda i: (i, 0)),
        out_shape=jax.ShapeDtypeStruct((_NV, _DV), jnp.float32),
    )(xv)
    return out.reshape(_N, _D)


# speedup vs baseline: 16.1809x; 1.3682x over previous
import jax
import jax.numpy as jnp
from jax.experimental import pallas as pl

_N = 1_000_000
_D = 64
_BLOCK = 20_000


def _copy_body(x_ref, o_ref):
    o_ref[...] = x_ref[...]

    @pl.when(pl.program_id(0) == 0)
    def _():
        two = jnp.full((1, _D), 2.0, jnp.float32)
        o_ref[pl.ds(1, 1), :] = two
        o_ref[pl.ds(3, 1), :] = two


def kernel(x):
    return pl.pallas_call(
        _copy_body,
        grid=(_N // _BLOCK,),
        in_specs=[pl.BlockSpec((_BLOCK, _D), lambda i: (i, 0))],
        out_specs=pl.BlockSpec((_BLOCK, _D), lambda i: (i, 0)),
        out_shape=jax.ShapeDtypeStruct((_N, _D), jnp.float32),
    )(x)
